# baseline (speedup 1.0000x reference)
import functools

import jax
import jax.numpy as jnp
from jax import lax
from jax.experimental import pallas as pl
from jax.experimental.pallas import tpu as pltpu
from jax.experimental.pallas import tpu_sc as plsc

B = 16384
H = 10
NCARD = 5
NSTATE = 3
CARD_EMB = 64
STATE_EMB = 32
HID = 64
OUT_DIM = 12
DENSE_IN = 3 + 6 + 2 * H + 2

NC, NS = 2, 16
NW = NC * NS
BPW = B // NW
GRP = BPW // 16
STR = 65


P_ROWS = 560


def _precompute_body(ct_ref, st_ref, w1e_ref, w1s_ref, p_ref):
    ct = ct_ref[...]
    rid = lax.broadcasted_iota(jnp.int32, ct.shape, 0)
    ct0 = jnp.where(rid == 0, 0.0, ct)
    S = jnp.dot(st_ref[...], w1s_ref[...], preferred_element_type=jnp.float32)
    T = [jnp.dot(ct0, w1e_ref[h], preferred_element_type=jnp.float32)
         for h in range(H)]
    zpad = jnp.zeros((3, 64), jnp.float32)
    t1p, t4p, t7p = (jnp.concatenate([T[k], zpad], axis=0) for k in (1, 4, 7))
    t9p = jnp.concatenate([T[9], zpad], axis=0)
    for s in range(NSTATE):
        for c0 in range(NCARD):
            p_ref[pl.ds(s * 40 + c0 * 8, 8), :] = (
                t1p + S[s: s + 1] + T[0][c0: c0 + 1])
    for c2 in range(NCARD):
        for c3 in range(NCARD):
            p_ref[pl.ds(120 + c2 * 40 + c3 * 8, 8), :] = (
                t4p + T[2][c2: c2 + 1] + T[3][c3: c3 + 1])
    for c5 in range(NCARD):
        for c6 in range(NCARD):
            p_ref[pl.ds(320 + c5 * 40 + c6 * 8, 8), :] = (
                t7p + T[5][c5: c5 + 1] + T[6][c6: c6 + 1])
    for c8 in range(NCARD):
        p_ref[pl.ds(520 + c8 * 8, 8), :] = t9p + T[8][c8: c8 + 1]


def _precompute_P(card_table, state_table, w1_emb, w1_state):
    return pl.pallas_call(
        _precompute_body,
        out_shape=jax.ShapeDtypeStruct((P_ROWS, 64), jnp.float32),
    )(card_table, state_table, w1_emb, w1_state)


def _bag_body(p_hbm, sidx_hbm, cidx_hbm, out_hbm, p_v, sidx_v, cidx_v, out_v):
    wid = lax.axis_index("s") * NC + lax.axis_index("c")
    base = wid * BPW
    pltpu.sync_copy(p_hbm, p_v)
    pltpu.sync_copy(sidx_hbm.at[pl.ds(base, BPW)], sidx_v)
    for h in range(H):
        pltpu.sync_copy(cidx_hbm.at[pl.ds(h * B + base, BPW)], cidx_v.at[h])

    lane = lax.iota(jnp.int32, 16)

    def group(g, _):
        s = g * 16
        sv = sidx_v[pl.ds(s, 16)]
        cv = [cidx_v[h, pl.ds(s, 16)] for h in range(H)]
        rows = [
            (sv * 40 + cv[0] * 8 + cv[1]) * STR,
            (cv[2] * 40 + cv[3] * 8 + cv[4] + 120) * STR,
            (cv[5] * 40 + cv[6] * 8 + cv[7] + 320) * STR,
            (cv[8] * 8 + cv[9] + 520) * STR,
        ]
        obase = (lane + s) * STR
        for d in range(64):
            acc = plsc.load_gather(p_v, [rows[0] + d])
            for r in rows[1:]:
                acc = acc + plsc.load_gather(p_v, [r + d])
            plsc.store_scatter(out_v, [obase + d], acc)
        return 0

    lax.fori_loop(0, GRP, group, 0)
    pltpu.sync_copy(out_v, out_hbm.at[pl.ds(base * STR, BPW * STR)])


@functools.lru_cache(maxsize=1)
def _make_bag_kernel():
    return functools.partial(
        pl.kernel,
        out_type=jax.ShapeDtypeStruct((B * STR,), jnp.float32),
        mesh=plsc.VectorSubcoreMesh(core_axis_name="c", subcore_axis_name="s"),
        compiler_params=pltpu.CompilerParams(needs_layout_passes=False),
        scratch_types=[
            pltpu.VMEM((P_ROWS * STR,), jnp.float32),
            pltpu.VMEM((BPW,), jnp.int32),
            pltpu.VMEM((H, BPW), jnp.int32),
            pltpu.VMEM((BPW * STR,), jnp.float32),
        ],
    )(_bag_body)


def _mlp_body(ch_ref, mo_ref, cs_ref, en_ref, bag_ref,
              w1_ref, b1_ref, w2_ref, b2_ref, w3_ref, b3_ref, out_ref):
    x = jnp.concatenate(
        [ch_ref[...], mo_ref[...], cs_ref[...], en_ref[...]], axis=1)
    h1 = jnp.dot(x, w1_ref[...], preferred_element_type=jnp.float32)
    h1 = jnp.maximum(h1 + bag_ref[:, :HID] + b1_ref[...], 0.0)
    h2 = jnp.dot(h1, w2_ref[...], preferred_element_type=jnp.float32)
    h2 = jnp.maximum(h2 + b2_ref[...], 0.0)
    out_ref[...] = jnp.dot(h2, w3_ref[...],
                           preferred_element_type=jnp.float32) + b3_ref[...]


def _mlp(character, monsters, cs2, energy, bag, w1d, b1, w2, b2, w3, b3,
         blk=2048):
    grid = (B // blk,)
    bspec = lambda w: pl.BlockSpec((blk, w), lambda i: (i, 0))
    wspec = lambda a, b: pl.BlockSpec((a, b), lambda i: (0, 0))
    return pl.pallas_call(
        _mlp_body,
        grid=grid,
        in_specs=[
            bspec(3), bspec(6), bspec(2 * H), bspec(2), bspec(STR),
            wspec(DENSE_IN, HID), wspec(1, HID),
            wspec(HID, HID), wspec(1, HID),
            wspec(HID, OUT_DIM), wspec(1, OUT_DIM),
        ],
        out_specs=bspec(OUT_DIM),
        out_shape=jax.ShapeDtypeStruct((B, OUT_DIM), jnp.float32),
    )(character, monsters, cs2, energy, bag, w1d, b1, w2, b2, w3, b3)


@jax.jit
def kernel(state_idx, character, monsters, card_idx, card_scalars, energy,
           card_table, state_table, W1, b1, W2, b2, W3, b3):
    w1_state = W1[0:STATE_EMB]
    w1_emb = jnp.stack(
        [W1[43 + 66 * h: 43 + 66 * h + CARD_EMB] for h in range(H)])
    dense_rows = ([32, 33, 34, 35, 36, 37, 38, 39, 40]
                  + [41 + 66 * h + s for h in range(H) for s in range(2)]
                  + [701, 702])
    w1d = W1[jnp.array(dense_rows)]

    P = _precompute_P(card_table, state_table, w1_emb, w1_state)

    bag = _make_bag_kernel()(
        jnp.pad(P, ((0, 0), (0, STR - 64))).reshape(-1),
        state_idx.astype(jnp.int32),
        jnp.transpose(card_idx.astype(jnp.int32)).reshape(-1)).reshape(B, STR)

    cs2 = card_scalars.reshape(B, 2 * H)
    out = _mlp(character, monsters, cs2, energy, bag,
               w1d, b1.reshape(1, HID), W2, b2.reshape(1, HID),
               W3, b3.reshape(1, OUT_DIM))
    return out


# speedup vs baseline: 1.2671x; 1.2671x over previous
import functools

import jax
import jax.numpy as jnp
from jax import lax
from jax.experimental import pallas as pl
from jax.experimental.pallas import tpu as pltpu
from jax.experimental.pallas import tpu_sc as plsc

B = 16384
H = 10
NCARD = 5
NSTATE = 3
CARD_EMB = 64
STATE_EMB = 32
HID = 64
OUT_DIM = 12
DENSE_IN = 3 + 6 + 2 * H + 2

NC, NS = 2, 16
NW = NC * NS
BPW = B // NW
GRP = BPW // 16
STR = 65


P_ROWS = 560


def _precompute_body(ct_ref, st_ref, w1e_ref, w1s_ref, p_ref):
    ct = ct_ref[...]
    rid = lax.broadcasted_iota(jnp.int32, ct.shape, 0)
    ct0 = jnp.where(rid == 0, 0.0, ct)
    S = jnp.dot(st_ref[...], w1s_ref[...], preferred_element_type=jnp.float32)
    T = [jnp.dot(ct0, w1e_ref[h], preferred_element_type=jnp.float32)
         for h in range(H)]
    zpad = jnp.zeros((3, 64), jnp.float32)
    t1p, t4p, t7p = (jnp.concatenate([T[k], zpad], axis=0) for k in (1, 4, 7))
    t9p = jnp.concatenate([T[9], zpad], axis=0)
    for s in range(NSTATE):
        for c0 in range(NCARD):
            p_ref[pl.ds(s * 40 + c0 * 8, 8), :] = (
                t1p + S[s: s + 1] + T[0][c0: c0 + 1])
    for c2 in range(NCARD):
        for c3 in range(NCARD):
            p_ref[pl.ds(120 + c2 * 40 + c3 * 8, 8), :] = (
                t4p + T[2][c2: c2 + 1] + T[3][c3: c3 + 1])
    for c5 in range(NCARD):
        for c6 in range(NCARD):
            p_ref[pl.ds(320 + c5 * 40 + c6 * 8, 8), :] = (
                t7p + T[5][c5: c5 + 1] + T[6][c6: c6 + 1])
    for c8 in range(NCARD):
        p_ref[pl.ds(520 + c8 * 8, 8), :] = t9p + T[8][c8: c8 + 1]


def _precompute_P(card_table, state_table, w1_emb, w1_state):
    return pl.pallas_call(
        _precompute_body,
        out_shape=jax.ShapeDtypeStruct((P_ROWS, 64), jnp.float32),
    )(card_table, state_table, w1_emb, w1_state)


def _bag_body(p_hbm, sidx_hbm, cidx_hbm, out_hbm, p_v, sidx_v, cidx_v, out_v,
              sem):
    wid = lax.axis_index("s") * NC + lax.axis_index("c")
    base = wid * BPW
    cps = [pltpu.async_copy(p_hbm, p_v, sem),
           pltpu.async_copy(sidx_hbm.at[pl.ds(base, BPW)], sidx_v, sem)]
    for h in range(H):
        cps.append(pltpu.async_copy(
            cidx_hbm.at[pl.ds(h * B + base, BPW)], cidx_v.at[h], sem))
    for cp in cps:
        cp.wait()

    lane = lax.iota(jnp.int32, 16)

    def group(g, _):
        s = g * 16
        sv = sidx_v[pl.ds(s, 16)]
        cv = [cidx_v[h, pl.ds(s, 16)] for h in range(H)]
        rows = [
            (sv * 40 + cv[0] * 8 + cv[1]) * STR,
            (cv[2] * 40 + cv[3] * 8 + cv[4] + 120) * STR,
            (cv[5] * 40 + cv[6] * 8 + cv[7] + 320) * STR,
            (cv[8] * 8 + cv[9] + 520) * STR,
        ]
        obase = (lane + s) * STR
        for d in range(64):
            acc = plsc.load_gather(p_v, [rows[0] + d])
            for r in rows[1:]:
                acc = acc + plsc.load_gather(p_v, [r + d])
            plsc.store_scatter(out_v, [obase + d], acc)
        return 0

    lax.fori_loop(0, GRP, group, 0)
    pltpu.sync_copy(out_v, out_hbm.at[pl.ds(base * STR, BPW * STR)])


@functools.lru_cache(maxsize=1)
def _make_bag_kernel():
    return functools.partial(
        pl.kernel,
        out_type=jax.ShapeDtypeStruct((B * STR,), jnp.float32),
        mesh=plsc.VectorSubcoreMesh(core_axis_name="c", subcore_axis_name="s"),
        compiler_params=pltpu.CompilerParams(needs_layout_passes=False),
        scratch_types=[
            pltpu.VMEM((P_ROWS * STR,), jnp.float32),
            pltpu.VMEM((BPW,), jnp.int32),
            pltpu.VMEM((H, BPW), jnp.int32),
            pltpu.VMEM((BPW * STR,), jnp.float32),
            pltpu.SemaphoreType.DMA,
        ],
    )(_bag_body)


def _mlp_body(x_ref, bag_ref,
              w1_ref, b1_ref, w2_ref, b2_ref, w3_ref, b3_ref, out_ref):
    h1 = jnp.dot(x_ref[...], w1_ref[...], preferred_element_type=jnp.float32)
    h1 = jnp.maximum(h1 + bag_ref[:, :HID] + b1_ref[...], 0.0)
    h2 = jnp.dot(h1, w2_ref[...], preferred_element_type=jnp.float32)
    h2 = jnp.maximum(h2 + b2_ref[...], 0.0)
    out_ref[...] = jnp.dot(h2, w3_ref[...],
                           preferred_element_type=jnp.float32) + b3_ref[...]


def _mlp(x, bag, w1d, b1, w2, b2, w3, b3, blk=4096):
    grid = (B // blk,)
    bspec = lambda w: pl.BlockSpec((blk, w), lambda i: (i, 0))
    wspec = lambda a, b: pl.BlockSpec((a, b), lambda i: (0, 0))
    return pl.pallas_call(
        _mlp_body,
        grid=grid,
        in_specs=[
            bspec(DENSE_IN), bspec(STR),
            wspec(DENSE_IN, HID), wspec(1, HID),
            wspec(HID, HID), wspec(1, HID),
            wspec(HID, OUT_DIM), wspec(1, OUT_DIM),
        ],
        out_specs=bspec(OUT_DIM),
        out_shape=jax.ShapeDtypeStruct((B, OUT_DIM), jnp.float32),
    )(x, bag, w1d, b1, w2, b2, w3, b3)


@jax.jit
def kernel(state_idx, character, monsters, card_idx, card_scalars, energy,
           card_table, state_table, W1, b1, W2, b2, W3, b3):
    w1_state = W1[0:STATE_EMB]
    w1_emb = jnp.stack(
        [W1[43 + 66 * h: 43 + 66 * h + CARD_EMB] for h in range(H)])
    dense_rows = ([32, 33, 34, 35, 36, 37, 38, 39, 40]
                  + [41 + 66 * h + s for h in range(H) for s in range(2)]
                  + [701, 702])
    w1d = W1[jnp.array(dense_rows)]

    P = _precompute_P(card_table, state_table, w1_emb, w1_state)

    bag = _make_bag_kernel()(
        jnp.pad(P, ((0, 0), (0, STR - 64))).reshape(-1),
        state_idx.astype(jnp.int32),
        jnp.transpose(card_idx.astype(jnp.int32)).reshape(-1)).reshape(B, STR)

    x = jnp.concatenate(
        [character, monsters, card_scalars.reshape(B, 2 * H), energy], axis=1)
    out = _mlp(x, bag,
               w1d, b1.reshape(1, HID), W2, b2.reshape(1, HID),
               W3, b3.reshape(1, OUT_DIM))
    return out


# speedup vs baseline: 1.3595x; 1.0729x over previous
import functools

import jax
import jax.numpy as jnp
from jax import lax
from jax.experimental import pallas as pl
from jax.experimental.pallas import tpu as pltpu
from jax.experimental.pallas import tpu_sc as plsc

B = 16384
H = 10
NCARD = 5
NSTATE = 3
CARD_EMB = 64
STATE_EMB = 32
HID = 64
OUT_DIM = 12
DENSE_IN = 3 + 6 + 2 * H + 2

NC, NS = 2, 16
NW = NC * NS
BPW = B // NW
GRP = BPW // 16
STR = 65


P_ROWS = 1800
PW = 33


def _pack_rows(v):
    bits = lax.bitcast_convert_type(v, jnp.int32) + 0x8000
    lo = lax.shift_right_logical(bits[:, :32], 16)
    hi = jnp.bitwise_and(bits[:, 32:], jnp.int32(-65536))
    return jnp.bitwise_or(lo, hi)


def _precompute_body(ct_ref, st_ref, w1e_ref, w1s_ref, p_ref):
    ct = ct_ref[...]
    rid = lax.broadcasted_iota(jnp.int32, ct.shape, 0)
    ct0 = jnp.where(rid == 0, 0.0, ct)
    S = jnp.dot(st_ref[...], w1s_ref[...], preferred_element_type=jnp.float32)
    T = [jnp.dot(ct0, w1e_ref[h], preferred_element_type=jnp.float32)
         for h in range(H)]
    zpad = jnp.zeros((3, 64), jnp.float32)
    t2p, t6p, t9p = (jnp.concatenate([T[k], zpad], axis=0) for k in (2, 6, 9))
    for s in range(NSTATE):
        for c0 in range(NCARD):
            for c1 in range(NCARD):
                base = ((s * 5 + c0) * 5 + c1) * 8
                p_ref[pl.ds(base, 8), :32] = _pack_rows(
                    t2p + (S[s: s + 1] + T[0][c0: c0 + 1] + T[1][c1: c1 + 1]))
    for c3 in range(NCARD):
        for c4 in range(NCARD):
            for c5 in range(NCARD):
                base = 600 + ((c3 * 5 + c4) * 5 + c5) * 8
                p_ref[pl.ds(base, 8), :32] = _pack_rows(
                    t6p + (T[3][c3: c3 + 1] + T[4][c4: c4 + 1]
                           + T[5][c5: c5 + 1]))
    for c7 in range(NCARD):
        for c8 in range(NCARD):
            base = 1600 + (c7 * 5 + c8) * 8
            p_ref[pl.ds(base, 8), :32] = _pack_rows(
                t9p + (T[7][c7: c7 + 1] + T[8][c8: c8 + 1]))


def _precompute_P(card_table, state_table, w1_emb, w1_state):
    return pl.pallas_call(
        _precompute_body,
        out_shape=jax.ShapeDtypeStruct((P_ROWS, PW), jnp.int32),
    )(card_table, state_table, w1_emb, w1_state)


def _bag_body(p_hbm, sidx_hbm, cidx_hbm, out_hbm, p_v, sidx_v, cidx_v, out_v,
              sem):
    wid = lax.axis_index("s") * NC + lax.axis_index("c")
    base = wid * BPW
    cps = [pltpu.async_copy(p_hbm, p_v, sem),
           pltpu.async_copy(sidx_hbm.at[pl.ds(base, BPW)], sidx_v, sem)]
    for h in range(H):
        cps.append(pltpu.async_copy(
            cidx_hbm.at[pl.ds(h * B + base, BPW)], cidx_v.at[h], sem))
    for cp in cps:
        cp.wait()

    lane = lax.iota(jnp.int32, 16)
    himask = jnp.int32(-65536)

    def group(g, _):
        s = g * 16
        sv = sidx_v[pl.ds(s, 16)]
        cv = [cidx_v[h, pl.ds(s, 16)] for h in range(H)]
        rows = [
            (((sv * 5 + cv[0]) * 5 + cv[1]) * 8 + cv[2]) * PW,
            ((((cv[3] * 5 + cv[4]) * 5 + cv[5]) * 8 + cv[6]) + 600) * PW,
            (((cv[7] * 5 + cv[8]) * 8 + cv[9]) + 1600) * PW,
        ]
        obase = (lane + s) * STR
        for j in range(32):
            w = [plsc.load_gather(p_v, [r + j]) for r in rows]
            acc_lo = sum(plsc.bitcast(lax.shift_left(x, 16), jnp.float32)
                         for x in w[1:]
                         ) + plsc.bitcast(lax.shift_left(w[0], 16), jnp.float32)
            acc_hi = sum(plsc.bitcast(jnp.bitwise_and(x, himask), jnp.float32)
                         for x in w[1:]
                         ) + plsc.bitcast(jnp.bitwise_and(w[0], himask),
                                          jnp.float32)
            plsc.store_scatter(out_v, [obase + j], acc_lo)
            plsc.store_scatter(out_v, [obase + j + 32], acc_hi)
        return 0

    lax.fori_loop(0, GRP, group, 0)
    pltpu.sync_copy(out_v, out_hbm.at[pl.ds(base * STR, BPW * STR)])


@functools.lru_cache(maxsize=1)
def _make_bag_kernel():
    return functools.partial(
        pl.kernel,
        out_type=jax.ShapeDtypeStruct((B * STR,), jnp.float32),
        mesh=plsc.VectorSubcoreMesh(core_axis_name="c", subcore_axis_name="s"),
        compiler_params=pltpu.CompilerParams(needs_layout_passes=False),
        scratch_types=[
            pltpu.VMEM((P_ROWS * PW,), jnp.int32),
            pltpu.VMEM((BPW,), jnp.int32),
            pltpu.VMEM((H, BPW), jnp.int32),
            pltpu.VMEM((BPW * STR,), jnp.float32),
            pltpu.SemaphoreType.DMA,
        ],
    )(_bag_body)


def _mlp_body(x_ref, bag_ref,
              w1_ref, b1_ref, w2_ref, b2_ref, w3_ref, b3_ref, out_ref):
    h1 = jnp.dot(x_ref[...], w1_ref[...], preferred_element_type=jnp.float32)
    h1 = jnp.maximum(h1 + bag_ref[:, :HID] + b1_ref[...], 0.0)
    h2 = jnp.dot(h1, w2_ref[...], preferred_element_type=jnp.float32)
    h2 = jnp.maximum(h2 + b2_ref[...], 0.0)
    out_ref[...] = jnp.dot(h2, w3_ref[...],
                           preferred_element_type=jnp.float32) + b3_ref[...]


def _mlp(x, bag, w1d, b1, w2, b2, w3, b3, blk=4096):
    grid = (B // blk,)
    bspec = lambda w: pl.BlockSpec((blk, w), lambda i: (i, 0))
    wspec = lambda a, b: pl.BlockSpec((a, b), lambda i: (0, 0))
    return pl.pallas_call(
        _mlp_body,
        grid=grid,
        in_specs=[
            bspec(DENSE_IN), bspec(STR),
            wspec(DENSE_IN, HID), wspec(1, HID),
            wspec(HID, HID), wspec(1, HID),
            wspec(HID, OUT_DIM), wspec(1, OUT_DIM),
        ],
        out_specs=bspec(OUT_DIM),
        out_shape=jax.ShapeDtypeStruct((B, OUT_DIM), jnp.float32),
    )(x, bag, w1d, b1, w2, b2, w3, b3)


@jax.jit
def kernel(state_idx, character, monsters, card_idx, card_scalars, energy,
           card_table, state_table, W1, b1, W2, b2, W3, b3):
    w1_state = W1[0:STATE_EMB]
    w1_emb = jnp.stack(
        [W1[43 + 66 * h: 43 + 66 * h + CARD_EMB] for h in range(H)])
    dense_rows = ([32, 33, 34, 35, 36, 37, 38, 39, 40]
                  + [41 + 66 * h + s for h in range(H) for s in range(2)]
                  + [701, 702])
    w1d = W1[jnp.array(dense_rows)]

    P = _precompute_P(card_table, state_table, w1_emb, w1_state)

    bag = _make_bag_kernel()(
        P.reshape(-1),
        state_idx.astype(jnp.int32),
        jnp.transpose(card_idx.astype(jnp.int32)).reshape(-1)).reshape(B, STR)

    x = jnp.concatenate(
        [character, monsters, card_scalars.reshape(B, 2 * H), energy], axis=1)
    out = _mlp(x, bag,
               w1d, b1.reshape(1, HID), W2, b2.reshape(1, HID),
               W3, b3.reshape(1, OUT_DIM))
    return out


# speedup vs baseline: 1.7435x; 1.2825x over previous
import functools

import jax
import jax.numpy as jnp
from jax import lax
from jax.experimental import pallas as pl
from jax.experimental.pallas import tpu as pltpu
from jax.experimental.pallas import tpu_sc as plsc

B = 16384
H = 10
NCARD = 5
NSTATE = 3
CARD_EMB = 64
STATE_EMB = 32
HID = 64
OUT_DIM = 12
DENSE_IN = 3 + 6 + 2 * H + 2

NC, NS = 2, 16
NW = NC * NS
BPW = B // NW
GRP = BPW // 16


P_ROWS = 1800
PW = 33


def _pack_rows(v):
    bits = lax.bitcast_convert_type(v, jnp.int32) + 0x8000
    lo = lax.shift_right_logical(bits[:, :32], 16)
    hi = jnp.bitwise_and(bits[:, 32:], jnp.int32(-65536))
    return jnp.bitwise_or(lo, hi)


def _precompute_body(ct_ref, st_ref, w1e_ref, w1s_ref, p_ref):
    ct = ct_ref[...]
    rid = lax.broadcasted_iota(jnp.int32, ct.shape, 0)
    ct0 = jnp.where(rid == 0, 0.0, ct)
    S = jnp.dot(st_ref[...], w1s_ref[...], preferred_element_type=jnp.float32)
    T = [jnp.dot(ct0, w1e_ref[h], preferred_element_type=jnp.float32)
         for h in range(H)]
    zpad = jnp.zeros((3, 64), jnp.float32)
    t2p, t6p, t9p = (jnp.concatenate([T[k], zpad], axis=0) for k in (2, 6, 9))
    for s in range(NSTATE):
        for c0 in range(NCARD):
            for c1 in range(NCARD):
                base = ((s * 5 + c0) * 5 + c1) * 8
                p_ref[pl.ds(base, 8), :32] = _pack_rows(
                    t2p + (S[s: s + 1] + T[0][c0: c0 + 1] + T[1][c1: c1 + 1]))
    for c3 in range(NCARD):
        for c4 in range(NCARD):
            for c5 in range(NCARD):
                base = 600 + ((c3 * 5 + c4) * 5 + c5) * 8
                p_ref[pl.ds(base, 8), :32] = _pack_rows(
                    t6p + (T[3][c3: c3 + 1] + T[4][c4: c4 + 1]
                           + T[5][c5: c5 + 1]))
    for c7 in range(NCARD):
        for c8 in range(NCARD):
            base = 1600 + (c7 * 5 + c8) * 8
            p_ref[pl.ds(base, 8), :32] = _pack_rows(
                t9p + (T[7][c7: c7 + 1] + T[8][c8: c8 + 1]))


def _precompute_P(card_table, state_table, w1_emb, w1_state):
    return pl.pallas_call(
        _precompute_body,
        out_shape=jax.ShapeDtypeStruct((P_ROWS, PW), jnp.int32),
    )(card_table, state_table, w1_emb, w1_state)


def _bag_body(p_hbm, sidx_hbm, cidx_hbm, out_hbm, p_v, sidx_v, cidx_v, out_v,
              sem):
    wid = lax.axis_index("s") * NC + lax.axis_index("c")
    base = wid * BPW
    cps = [pltpu.async_copy(p_hbm, p_v, sem),
           pltpu.async_copy(sidx_hbm.at[pl.ds(base, BPW)], sidx_v, sem)]
    for h in range(H):
        cps.append(pltpu.async_copy(
            cidx_hbm.at[pl.ds(h * B + base, BPW)], cidx_v.at[h], sem))
    for cp in cps:
        cp.wait()

    himask = jnp.int32(-65536)

    def unpk(x):
        return (plsc.bitcast(lax.shift_left(x, 16), jnp.float32),
                plsc.bitcast(jnp.bitwise_and(x, himask), jnp.float32))

    def chunk(c):
        def group(g, _):
            s = c * (BPW // 2) + g * 16
            sv = sidx_v[pl.ds(s, 16)]
            cv = [cidx_v[h, pl.ds(s, 16)] for h in range(H)]
            rowv = [
                (((sv * 5 + cv[0]) * 5 + cv[1]) * 8 + cv[2]) * PW,
                ((((cv[3] * 5 + cv[4]) * 5 + cv[5]) * 8 + cv[6]) + 600) * PW,
                (((cv[7] * 5 + cv[8]) * 8 + cv[9]) + 1600) * PW,
            ]
            for u in range(16):
                offs = [r[u] for r in rowv]
                ob = (g * 16 + u) * 128
                lo = [unpk(p_v[pl.ds(o, 16)]) for o in offs]
                hi = [unpk(p_v[pl.ds(o + 16, 16)]) for o in offs]
                out_v[pl.ds(ob, 16)] = lo[0][0] + lo[1][0] + lo[2][0]
                out_v[pl.ds(ob + 16, 16)] = hi[0][0] + hi[1][0] + hi[2][0]
                out_v[pl.ds(ob + 32, 16)] = lo[0][1] + lo[1][1] + lo[2][1]
                out_v[pl.ds(ob + 48, 16)] = hi[0][1] + hi[1][1] + hi[2][1]
            return 0

        lax.fori_loop(0, GRP // 2, group, 0)
        pltpu.sync_copy(out_v, out_hbm.at[
            pl.ds((base + c * (BPW // 2)) * 128, (BPW // 2) * 128)])

    chunk(0)
    chunk(1)


@functools.lru_cache(maxsize=1)
def _make_bag_kernel():
    return functools.partial(
        pl.kernel,
        out_type=jax.ShapeDtypeStruct((B * 128,), jnp.float32),
        mesh=plsc.VectorSubcoreMesh(core_axis_name="c", subcore_axis_name="s"),
        compiler_params=pltpu.CompilerParams(needs_layout_passes=False),
        scratch_types=[
            pltpu.VMEM((P_ROWS * PW,), jnp.int32),
            pltpu.VMEM((BPW,), jnp.int32),
            pltpu.VMEM((H, BPW), jnp.int32),
            pltpu.VMEM(((BPW // 2) * 128,), jnp.float32),
            pltpu.SemaphoreType.DMA,
        ],
    )(_bag_body)


def _mlp_body(x_ref, bag_ref,
              w1_ref, b1_ref, w2_ref, b2_ref, w3_ref, b3_ref, out_ref):
    h1 = jnp.dot(x_ref[...], w1_ref[...], preferred_element_type=jnp.float32)
    h1 = jnp.maximum(h1 + bag_ref[:, :HID] + b1_ref[...], 0.0)
    h2 = jnp.dot(h1, w2_ref[...], preferred_element_type=jnp.float32)
    h2 = jnp.maximum(h2 + b2_ref[...], 0.0)
    out_ref[...] = jnp.dot(h2, w3_ref[...],
                           preferred_element_type=jnp.float32) + b3_ref[...]


def _mlp(x, bag, w1d, b1, w2, b2, w3, b3, blk=4096):
    grid = (B // blk,)
    bspec = lambda w: pl.BlockSpec((blk, w), lambda i: (i, 0))
    wspec = lambda a, b: pl.BlockSpec((a, b), lambda i: (0, 0))
    return pl.pallas_call(
        _mlp_body,
        grid=grid,
        in_specs=[
            bspec(DENSE_IN), bspec(128),
            wspec(DENSE_IN, HID), wspec(1, HID),
            wspec(HID, HID), wspec(1, HID),
            wspec(HID, OUT_DIM), wspec(1, OUT_DIM),
        ],
        out_specs=bspec(OUT_DIM),
        out_shape=jax.ShapeDtypeStruct((B, OUT_DIM), jnp.float32),
    )(x, bag, w1d, b1, w2, b2, w3, b3)


@jax.jit
def kernel(state_idx, character, monsters, card_idx, card_scalars, energy,
           card_table, state_table, W1, b1, W2, b2, W3, b3):
    w1_state = W1[0:STATE_EMB]
    w1_emb = jnp.stack(
        [W1[43 + 66 * h: 43 + 66 * h + CARD_EMB] for h in range(H)])
    dense_rows = ([32, 33, 34, 35, 36, 37, 38, 39, 40]
                  + [41 + 66 * h + s for h in range(H) for s in range(2)]
                  + [701, 702])
    w1d = W1[jnp.array(dense_rows)]

    P = _precompute_P(card_table, state_table, w1_emb, w1_state)

    bag = _make_bag_kernel()(
        P.reshape(-1),
        state_idx.astype(jnp.int32),
        jnp.transpose(card_idx.astype(jnp.int32)).reshape(-1)).reshape(B, 128)

    x = jnp.concatenate(
        [character, monsters, card_scalars.reshape(B, 2 * H), energy], axis=1)
    out = _mlp(x, bag,
               w1d, b1.reshape(1, HID), W2, b2.reshape(1, HID),
               W3, b3.reshape(1, OUT_DIM))
    return out


# speedup vs baseline: 1.7784x; 1.0200x over previous
import functools

import jax
import jax.numpy as jnp
from jax import lax
from jax.experimental import pallas as pl
from jax.experimental.pallas import tpu as pltpu
from jax.experimental.pallas import tpu_sc as plsc

B = 16384
H = 10
NCARD = 5
NSTATE = 3
CARD_EMB = 64
STATE_EMB = 32
HID = 64
OUT_DIM = 12
DENSE_IN = 3 + 6 + 2 * H + 2

NC, NS = 2, 16
NW = NC * NS
BPW = B // NW
GRP = BPW // 16


P_ROWS = 1800
PW = 33


def _pack_rows(v):
    bits = lax.bitcast_convert_type(v, jnp.int32) + 0x8000
    lo = lax.shift_right_logical(bits[:, :32], 16)
    hi = jnp.bitwise_and(bits[:, 32:], jnp.int32(-65536))
    return jnp.bitwise_or(lo, hi)


def _precompute_body(ct_ref, st_ref, w1e_ref, w1s_ref, p_ref):
    ct = ct_ref[...]
    rid = lax.broadcasted_iota(jnp.int32, ct.shape, 0)
    ct0 = jnp.where(rid == 0, 0.0, ct)
    S = jnp.dot(st_ref[...], w1s_ref[...], preferred_element_type=jnp.float32)
    T = [jnp.dot(ct0, w1e_ref[h], preferred_element_type=jnp.float32)
         for h in range(H)]
    zpad = jnp.zeros((3, 64), jnp.float32)
    t2p, t6p, t9p = (jnp.concatenate([T[k], zpad], axis=0) for k in (2, 6, 9))
    for s in range(NSTATE):
        for c0 in range(NCARD):
            for c1 in range(NCARD):
                base = ((s * 5 + c0) * 5 + c1) * 8
                p_ref[pl.ds(base, 8), :32] = _pack_rows(
                    t2p + (S[s: s + 1] + T[0][c0: c0 + 1] + T[1][c1: c1 + 1]))
    for c3 in range(NCARD):
        for c4 in range(NCARD):
            for c5 in range(NCARD):
                base = 600 + ((c3 * 5 + c4) * 5 + c5) * 8
                p_ref[pl.ds(base, 8), :32] = _pack_rows(
                    t6p + (T[3][c3: c3 + 1] + T[4][c4: c4 + 1]
                           + T[5][c5: c5 + 1]))
    for c7 in range(NCARD):
        for c8 in range(NCARD):
            base = 1600 + (c7 * 5 + c8) * 8
            p_ref[pl.ds(base, 8), :32] = _pack_rows(
                t9p + (T[7][c7: c7 + 1] + T[8][c8: c8 + 1]))


def _precompute_P(card_table, state_table, w1_emb, w1_state):
    return pl.pallas_call(
        _precompute_body,
        out_shape=jax.ShapeDtypeStruct((P_ROWS, PW), jnp.int32),
    )(card_table, state_table, w1_emb, w1_state)


def _bag_body(p_hbm, sidx_hbm, cidx_hbm, out_hbm, p_v, sidx_v, cidx_v, out_v,
              sem):
    wid = lax.axis_index("s") * NC + lax.axis_index("c")
    base = wid * BPW
    cps = [pltpu.async_copy(p_hbm, p_v, sem),
           pltpu.async_copy(sidx_hbm.at[pl.ds(base, BPW)], sidx_v, sem)]
    for h in range(H):
        cps.append(pltpu.async_copy(
            cidx_hbm.at[pl.ds(h * B + base, BPW)], cidx_v.at[h], sem))
    for cp in cps:
        cp.wait()

    himask = jnp.int32(-65536)

    def unpk(x):
        return (plsc.bitcast(lax.shift_left(x, 16), jnp.float32),
                plsc.bitcast(jnp.bitwise_and(x, himask), jnp.float32))

    def chunk(c):
        def group(g, _):
            s = c * (BPW // 2) + g * 16
            sv = sidx_v[pl.ds(s, 16)]
            cv = [cidx_v[h, pl.ds(s, 16)] for h in range(H)]
            rowv = [
                (((sv * 5 + cv[0]) * 5 + cv[1]) * 8 + cv[2]) * PW,
                ((((cv[3] * 5 + cv[4]) * 5 + cv[5]) * 8 + cv[6]) + 600) * PW,
                (((cv[7] * 5 + cv[8]) * 8 + cv[9]) + 1600) * PW,
            ]
            for u in range(16):
                offs = [r[u] for r in rowv]
                ob = (g * 16 + u) * 128
                lo = [unpk(p_v[pl.ds(o, 16)]) for o in offs]
                hi = [unpk(p_v[pl.ds(o + 16, 16)]) for o in offs]
                out_v[pl.ds(ob, 16)] = lo[0][0] + lo[1][0] + lo[2][0]
                out_v[pl.ds(ob + 16, 16)] = hi[0][0] + hi[1][0] + hi[2][0]
                out_v[pl.ds(ob + 32, 16)] = lo[0][1] + lo[1][1] + lo[2][1]
                out_v[pl.ds(ob + 48, 16)] = hi[0][1] + hi[1][1] + hi[2][1]
            return 0

        lax.fori_loop(0, GRP // 2, group, 0)
        pltpu.sync_copy(out_v, out_hbm.at[
            pl.ds((base + c * (BPW // 2)) * 128, (BPW // 2) * 128)])

    chunk(0)
    chunk(1)


@functools.lru_cache(maxsize=1)
def _make_bag_kernel():
    return functools.partial(
        pl.kernel,
        out_type=jax.ShapeDtypeStruct((B * 128,), jnp.float32),
        mesh=plsc.VectorSubcoreMesh(core_axis_name="c", subcore_axis_name="s"),
        compiler_params=pltpu.CompilerParams(needs_layout_passes=False),
        scratch_types=[
            pltpu.VMEM((P_ROWS * PW,), jnp.int32),
            pltpu.VMEM((BPW,), jnp.int32),
            pltpu.VMEM((H, BPW), jnp.int32),
            pltpu.VMEM(((BPW // 2) * 128,), jnp.float32),
            pltpu.SemaphoreType.DMA,
        ],
    )(_bag_body)


def _mlp_body(x_ref, bag_ref,
              w1_ref, b1_ref, w2_ref, b2_ref, w3_ref, b3_ref, out_ref):
    h1 = jnp.dot(x_ref[...], w1_ref[...], preferred_element_type=jnp.float32)
    h1 = jnp.maximum(h1 + bag_ref[:, :HID] + b1_ref[...], 0.0)
    h2 = jnp.dot(h1, w2_ref[...], preferred_element_type=jnp.float32)
    h2 = jnp.maximum(h2 + b2_ref[...], 0.0)
    out_ref[...] = jnp.dot(h2, w3_ref[...],
                           preferred_element_type=jnp.float32) + b3_ref[...]


def _mlp(x, bag, w1d, b1, w2, b2, w3, b3, blk=8192):
    grid = (B // blk,)
    bspec = lambda w: pl.BlockSpec((blk, w), lambda i: (i, 0))
    wspec = lambda a, b: pl.BlockSpec((a, b), lambda i: (0, 0))
    return pl.pallas_call(
        _mlp_body,
        grid=grid,
        in_specs=[
            bspec(DENSE_IN), bspec(128),
            wspec(DENSE_IN, HID), wspec(1, HID),
            wspec(HID, HID), wspec(1, HID),
            wspec(HID, OUT_DIM), wspec(1, OUT_DIM),
        ],
        out_specs=bspec(OUT_DIM),
        out_shape=jax.ShapeDtypeStruct((B, OUT_DIM), jnp.float32),
    )(x, bag, w1d, b1, w2, b2, w3, b3)


@jax.jit
def kernel(state_idx, character, monsters, card_idx, card_scalars, energy,
           card_table, state_table, W1, b1, W2, b2, W3, b3):
    w1_state = W1[0:STATE_EMB]
    w1_emb = jnp.stack(
        [W1[43 + 66 * h: 43 + 66 * h + CARD_EMB] for h in range(H)])
    dense_rows = ([32, 33, 34, 35, 36, 37, 38, 39, 40]
                  + [41 + 66 * h + s for h in range(H) for s in range(2)]
                  + [701, 702])
    w1d = W1[jnp.array(dense_rows)]

    P = _precompute_P(card_table, state_table, w1_emb, w1_state)

    bag = _make_bag_kernel()(
        P.reshape(-1),
        state_idx.astype(jnp.int32),
        jnp.transpose(card_idx.astype(jnp.int32)).reshape(-1)).reshape(B, 128)

    x = jnp.concatenate(
        [character, monsters, card_scalars.reshape(B, 2 * H), energy], axis=1)
    out = _mlp(x, bag,
               w1d, b1.reshape(1, HID), W2, b2.reshape(1, HID),
               W3, b3.reshape(1, OUT_DIM))
    return out
